# baseline (speedup 1.0000x reference)
import functools

import jax
import jax.numpy as jnp
from jax import lax
from jax.experimental import pallas as pl
from jax.experimental.pallas import tpu as pltpu
from jax.experimental.pallas import tpu_sc as plsc

N = 10000
NPAD = 10240
D = 128
H = 128
OUT = 64
E = 320000
NTILES = 32
CH = 32
PER_TILE = 10240
CHUNKS = PER_TILE // CH
EPAD = PER_TILE * NTILES
DUMMY = NPAD - 1
DEN_ROWS = NPAD // 128
NROWS = NPAD + 128
RB = 512
GRID = NPAD // RB
RPT = NROWS // 16


def _tc_pre_body(x_ref, w_ref, as_ref, ad_ref, h_ref, hq_ref, aux_ref):
    h = jnp.dot(x_ref[...], w_ref[...], preferred_element_type=jnp.float32)
    h_ref[...] = h
    asrc = jnp.sum(h * as_ref[...], axis=1, keepdims=True)
    adst = jnp.sum(h * ad_ref[...], axis=1, keepdims=True)
    ssum = asrc + adst
    wself = jnp.exp(jnp.where(ssum >= 0.0, ssum, 0.2 * ssum))
    rmax = jnp.maximum(jnp.max(jnp.abs(h), axis=1, keepdims=True), 1e-30)
    scale = rmax * (1.0 / 32600.0)
    q = jnp.round(h * (1.0 / scale)).astype(jnp.int32)
    lo = lax.bitwise_and(q[:, 0:64], 0xFFFF)
    hq_ref[...] = lax.bitwise_or(lo, lax.shift_left(q[:, 64:128], 16))
    lane = lax.broadcasted_iota(jnp.int32, (RB, H), 1)
    aux_ref[...] = (jnp.where(lane == 0, asrc, 0.0)
                    + jnp.where(lane == 1, adst, 0.0)
                    + jnp.where(lane == 2, wself, 0.0)
                    + jnp.where(lane == 3, scale, 0.0))


def _tc_pre(x, w, a_s, a_d):
    return pl.pallas_call(
        _tc_pre_body,
        grid=(GRID,),
        in_specs=[
            pl.BlockSpec((RB, D), lambda i: (i, 0)),
            pl.BlockSpec((D, H), lambda i: (0, 0)),
            pl.BlockSpec((1, H), lambda i: (0, 0)),
            pl.BlockSpec((1, H), lambda i: (0, 0)),
        ],
        out_specs=[
            pl.BlockSpec((RB, H), lambda i: (i, 0)),
            pl.BlockSpec((RB, H // 2), lambda i: (i, 0)),
            pl.BlockSpec((RB, H), lambda i: (i, 0)),
        ],
        out_shape=[
            jax.ShapeDtypeStruct((NPAD, H), jnp.float32),
            jax.ShapeDtypeStruct((NPAD, H // 2), jnp.int32),
            jax.ShapeDtypeStruct((NPAD, H), jnp.float32),
        ],
    )(x, w, a_s.reshape(1, H), a_d.reshape(1, H))


def _tc_combine_body(f_ref, d_ref, h_ref, aux_ref, b_ref, x_ref):
    i = pl.program_id(0)
    acc = f_ref[0] + f_ref[1]
    den = d_ref[0] + d_ref[1]
    lane = lax.broadcasted_iota(jnp.int32, (RB, H), 1)
    aux = aux_ref[...]
    wself = jnp.sum(jnp.where(lane == 2, aux, 0.0), axis=1, keepdims=True)
    num = acc + wself * h_ref[...]
    xv = num / (den + wself + 1e-16) + b_ref[...]
    xv = jnp.maximum(xv, 0.0)
    rid = i * RB + lax.broadcasted_iota(jnp.int32, (RB, H), 0)
    x_ref[...] = jnp.where(rid < N, xv, 0.0)


def _tc_combine(feats, den, h, aux, b):
    return pl.pallas_call(
        _tc_combine_body,
        grid=(GRID,),
        in_specs=[
            pl.BlockSpec((2, RB, H), lambda i: (0, i, 0)),
            pl.BlockSpec((2, RB, 1), lambda i: (0, i, 0)),
            pl.BlockSpec((RB, H), lambda i: (i, 0)),
            pl.BlockSpec((RB, H), lambda i: (i, 0)),
            pl.BlockSpec((1, H), lambda i: (0, 0)),
        ],
        out_specs=pl.BlockSpec((RB, H), lambda i: (i, 0)),
        out_shape=jax.ShapeDtypeStruct((NPAD, H), jnp.float32),
    )(feats, den, h, aux, b.reshape(1, H))


FRB = 400


def _tc_final_body(x1_ref, x2_ref, x3_ref, w_ref, b_ref, o_ref):
    xjk = jnp.maximum(jnp.maximum(x1_ref[...], x2_ref[...]), x3_ref[...])
    o_ref[...] = (jnp.dot(xjk, w_ref[...], preferred_element_type=jnp.float32)
                  + b_ref[...])


def _tc_final(x1, x2, x3, w_out, b_out):
    return pl.pallas_call(
        _tc_final_body,
        grid=(N // FRB,),
        in_specs=[
            pl.BlockSpec((FRB, H), lambda i: (i, 0)),
            pl.BlockSpec((FRB, H), lambda i: (i, 0)),
            pl.BlockSpec((FRB, H), lambda i: (i, 0)),
            pl.BlockSpec((H, OUT), lambda i: (0, 0)),
            pl.BlockSpec((1, OUT), lambda i: (0, 0)),
        ],
        out_specs=pl.BlockSpec((FRB, OUT), lambda i: (i, 0)),
        out_shape=jax.ShapeDtypeStruct((N, OUT), jnp.float32),
    )(x1, x2, x3, w_out, b_out.reshape(1, OUT))


_SC_MESH = plsc.VectorSubcoreMesh(core_axis_name="c", subcore_axis_name="s")

_SPLAT_DNUMS = lax.GatherDimensionNumbers(
    offset_dims=(), collapsed_slice_dims=(0,), start_index_map=(0,))


def _splat(vec16, k):
    idx = jnp.full((16, 1), k, jnp.int32)
    return lax.gather(vec16, idx, _SPLAT_DNUMS, (1,),
                      mode=lax.GatherScatterMode.PROMISE_IN_BOUNDS)


@functools.partial(
    pl.kernel,
    mesh=_SC_MESH,
    compiler_params=pltpu.CompilerParams(use_tc_tiling_on_sc=False),
    out_type=jax.ShapeDtypeStruct((2, NROWS, H), jnp.float32),
    scratch_types=(
        [pltpu.VMEM((CH,), jnp.int32) for _ in range(8)]
        + [pltpu.VMEM((CH,), jnp.int32) for _ in range(8)]
        + [
            pltpu.VMEM((CH,), jnp.float32),
            pltpu.VMEM((CH,), jnp.float32),
            pltpu.VMEM((CH,), jnp.float32),
            pltpu.VMEM((CH,), jnp.float32),
            pltpu.VMEM((CH,), jnp.float32),
            pltpu.VMEM((CH,), jnp.float32),
            pltpu.VMEM((CH,), jnp.float32),
            pltpu.VMEM((CH,), jnp.float32),
            pltpu.VMEM((CH,), jnp.float32),
            pltpu.VMEM((CH,), jnp.float32),
            pltpu.VMEM((CH,), jnp.float32),
            pltpu.VMEM((CH,), jnp.float32),
            pltpu.VMEM((CH, H // 2), jnp.int32),
            pltpu.VMEM((CH, H // 2), jnp.int32),
            pltpu.VMEM((CH, H // 2), jnp.int32),
            pltpu.VMEM((CH, H // 2), jnp.int32),
            pltpu.VMEM((CH, H), jnp.float32),
            pltpu.VMEM((CH, H), jnp.float32),
            pltpu.VMEM((DEN_ROWS, H), jnp.float32),
            pltpu.VMEM((DEN_ROWS,), jnp.int32),
            pltpu.VMEM_SHARED((NROWS, H), jnp.float32),
            pltpu.SemaphoreType.DMA,
            pltpu.SemaphoreType.DMA,
            pltpu.SemaphoreType.DMA,
        ]
    ),
)
def _sc_edges(hq_hbm, asrc_hbm, adst_hbm, scale_hbm, src_hbm, dst_hbm,
              zeros_hbm, denidx_hbm, out_hbm,
              srcr0, srcr1, srcr2, srcr3, srcr4, srcr5, srcr6, srcr7,
              dstr0, dstr1, dstr2, dstr3, dstr4, dstr5, dstr6, dstr7,
              as0, as1, as2, as3, ad0, ad1, ad2, ad3, sc0, sc1, sc2, sc3,
              rows0, rows1, rows2, rows3, scl0, scl1,
              den_l, denidx_v, acc_sh, isem, gsem, ssem):
    c = lax.axis_index("c")
    s = lax.axis_index("s")
    wid = c * 16 + s
    src_v = (srcr0, srcr1, srcr2, srcr3, srcr4, srcr5, srcr6, srcr7)
    dst_v = (dstr0, dstr1, dstr2, dstr3, dstr4, dstr5, dstr6, dstr7)
    asv_v = (as0, as1, as2, as3)
    adv_v = (ad0, ad1, ad2, ad3)
    scv_v = (sc0, sc1, sc2, sc3)
    rows_v = (rows0, rows1, rows2, rows3)
    scl_v = (scl0, scl1)

    pltpu.sync_copy(zeros_hbm.at[pl.ds(s * RPT, RPT)],
                    acc_sh.at[pl.ds(s * RPT, RPT)])
    pltpu.sync_copy(zeros_hbm.at[pl.ds(0, DEN_ROWS)], den_l)
    pltpu.sync_copy(denidx_hbm, denidx_v)
    plsc.subcore_barrier()

    lane = lax.iota(jnp.int32, 16)

    def issue_idx(r, g):
        base = wid * PER_TILE + g * CH
        pltpu.async_copy(src_hbm.at[pl.ds(base, CH)], src_v[r], isem)
        pltpu.async_copy(dst_hbm.at[pl.ds(base, CH)], dst_v[r], isem)

    def wait_idx(r, g):
        base = wid * PER_TILE + g * CH
        pltpu.make_async_copy(src_hbm.at[pl.ds(base, CH)], src_v[r], isem).wait()
        pltpu.make_async_copy(dst_hbm.at[pl.ds(base, CH)], dst_v[r], isem).wait()

    def issue_gathers(r, b):
        pltpu.async_copy(hq_hbm.at[src_v[r]], rows_v[b], gsem)
        pltpu.async_copy(asrc_hbm.at[src_v[r]], asv_v[b], gsem)
        pltpu.async_copy(adst_hbm.at[dst_v[r]], adv_v[b], gsem)
        pltpu.async_copy(scale_hbm.at[src_v[r]], scv_v[b], gsem)

    def wait_gathers(r, b):
        pltpu.make_async_copy(hq_hbm.at[src_v[r]], rows_v[b], gsem).wait()
        pltpu.make_async_copy(asrc_hbm.at[src_v[r]], asv_v[b], gsem).wait()
        pltpu.make_async_copy(adst_hbm.at[dst_v[r]], adv_v[b], gsem).wait()
        pltpu.make_async_copy(scale_hbm.at[src_v[r]], scv_v[b], gsem).wait()

    def wait_scatter(r, b):
        pltpu.make_async_copy(scl_v[b % 2], acc_sh.at[dst_v[r]], ssem).wait()

    def compute_chunk(r, b):
        bs = b % 2

        def group_body(t, carry):
            e = asv_v[b][pl.ds(t * 16, 16)] + adv_v[b][pl.ds(t * 16, 16)]
            e = jnp.where(e >= 0.0, e, 0.2 * e)
            w16 = jnp.exp(e)
            ws16 = w16 * scv_v[b][pl.ds(t * 16, 16)]
            d16 = dst_v[r][pl.ds(t * 16, 16)]
            for k in range(16):
                rr_ = t * 16 + k
                wvec = jnp.full((16,), w16[k], jnp.float32)
                wsvec = jnp.full((16,), ws16[k], jnp.float32)
                for j in range(H // 32):
                    v = rows_v[b][rr_, pl.ds(j * 16, 16)]
                    lo = lax.shift_right_arithmetic(lax.shift_left(v, 16), 16)
                    hi = lax.shift_right_arithmetic(v, 16)
                    scl_v[bs][rr_, pl.ds(j * 16, 16)] = (
                        lo.astype(jnp.float32) * wsvec)
                    scl_v[bs][rr_, pl.ds(64 + j * 16, 16)] = (
                        hi.astype(jnp.float32) * wsvec)
                d_s = d16[k]
                rr = lax.shift_right_logical(d_s, 7)
                bb = lax.bitwise_and(lax.shift_right_logical(d_s, 4), 7) * 16
                lk = lax.bitwise_and(d_s, 15)
                blk = den_l[rr, pl.ds(bb, 16)]
                den_l[rr, pl.ds(bb, 16)] = blk + jnp.where(lane == lk, wvec, 0.0)
            return carry

        lax.fori_loop(0, CH // 16, group_body, 0)

    def scatter_chunk(r, b):
        pltpu.async_copy(scl_v[b % 2], acc_sh.at[dst_v[r]], ssem, add=True)

    for g0 in range(4):
        issue_idx(g0, g0)
    wait_idx(0, 0)
    issue_gathers(0, 0)
    wait_idx(1, 1)
    issue_gathers(1, 1)

    def octo_body(i, carry):
        for q in range(8):
            g = 8 * i + q
            r, b = q, q % 4

            @pl.when(g >= 2)
            def _():
                wait_scatter((q - 2) % 8, (q - 2) % 4)

            @pl.when(g + 4 < CHUNKS)
            def _():
                issue_idx((q + 4) % 8, g + 4)

            @pl.when(g + 2 < CHUNKS)
            def _():
                wait_idx((q + 2) % 8, g + 2)
                issue_gathers((q + 2) % 8, (q + 2) % 4)

            wait_gathers(r, b)
            compute_chunk(r, b)
            scatter_chunk(r, b)
        return carry

    lax.fori_loop(0, CHUNKS // 8, octo_body, 0)
    wait_scatter((CHUNKS - 2) % 8, (CHUNKS - 2) % 4)
    wait_scatter((CHUNKS - 1) % 8, (CHUNKS - 1) % 4)

    pltpu.async_copy(den_l, acc_sh.at[denidx_v], ssem, add=True)
    pltpu.make_async_copy(den_l, acc_sh.at[denidx_v], ssem).wait()
    plsc.subcore_barrier()

    pltpu.sync_copy(acc_sh.at[pl.ds(s * RPT, RPT)],
                    out_hbm.at[c, pl.ds(s * RPT, RPT)])


def kernel(x, edge_index, W1, a1_s, a1_d, b1, W2, a2_s, a2_d, b2,
           W3, a3_s, a3_d, b3, W_out, b_out):
    ei = edge_index.astype(jnp.int32)
    src = jnp.concatenate([ei[0], jnp.zeros((EPAD - E,), jnp.int32)])
    dst = jnp.concatenate([ei[1], jnp.full((EPAD - E,), DUMMY, jnp.int32)])
    xp = jnp.pad(x, ((0, NPAD - N), (0, 0)))
    zeros = jnp.zeros((NROWS, H), jnp.float32)
    denidx = NPAD + jnp.arange(DEN_ROWS, dtype=jnp.int32)

    def gat_layer(xin, W, a_s, a_d, b):
        h, hq, aux = _tc_pre(xin, W, a_s, a_d)
        parts = _sc_edges(hq, aux[:, 0], aux[:, 1], aux[:, 3], src, dst,
                          zeros, denidx)
        feats = parts[:, :NPAD, :]
        den = parts[:, NPAD:NPAD + DEN_ROWS, :].reshape(2, NPAD, 1)
        return _tc_combine(feats, den, h, aux, b)

    x1 = gat_layer(xp, W1, a1_s, a1_d, b1)
    x2 = gat_layer(x1, W2, a2_s, a2_d, b2)
    x3 = gat_layer(x2, W3, a3_s, a3_d, b3)
    return _tc_final(x1, x2, x3, W_out, b_out)


# speedup vs baseline: 1.3286x; 1.3286x over previous
import functools

import jax
import jax.numpy as jnp
from jax import lax
from jax.experimental import pallas as pl
from jax.experimental.pallas import tpu as pltpu
from jax.experimental.pallas import tpu_sc as plsc

N = 10000
NPAD = 10240
D = 128
H = 128
OUT = 64
E = 320000
NTILES = 32
CH = 64
PER_TILE = 10240
CHUNKS = PER_TILE // CH
EPAD = PER_TILE * NTILES
DUMMY = NPAD - 1
DEN_ROWS = NPAD // 128
NROWS = NPAD + 128
RB = 512
GRID = NPAD // RB
RPT = NROWS // 16


def _tc_pre_body(x_ref, w_ref, as_ref, ad_ref, h_ref, aux_ref):
    h = jnp.dot(x_ref[...], w_ref[...], preferred_element_type=jnp.float32)
    h_ref[...] = h
    asrc = jnp.sum(h * as_ref[...], axis=1, keepdims=True)
    adst = jnp.sum(h * ad_ref[...], axis=1, keepdims=True)
    ssum = asrc + adst
    wself = jnp.exp(jnp.where(ssum >= 0.0, ssum, 0.2 * ssum))
    lane = lax.broadcasted_iota(jnp.int32, (RB, H), 1)
    aux_ref[...] = (jnp.where(lane == 0, asrc, 0.0)
                    + jnp.where(lane == 1, adst, 0.0)
                    + jnp.where(lane == 2, wself, 0.0))


def _tc_pre(x, w, a_s, a_d):
    return pl.pallas_call(
        _tc_pre_body,
        grid=(GRID,),
        in_specs=[
            pl.BlockSpec((RB, D), lambda i: (i, 0)),
            pl.BlockSpec((D, H), lambda i: (0, 0)),
            pl.BlockSpec((1, H), lambda i: (0, 0)),
            pl.BlockSpec((1, H), lambda i: (0, 0)),
        ],
        out_specs=[
            pl.BlockSpec((RB, H), lambda i: (i, 0)),
            pl.BlockSpec((RB, H), lambda i: (i, 0)),
        ],
        out_shape=[
            jax.ShapeDtypeStruct((NPAD, H), jnp.float32),
            jax.ShapeDtypeStruct((NPAD, H), jnp.float32),
        ],
    )(x, w, a_s.reshape(1, H), a_d.reshape(1, H))


def _tc_combine_body(f_ref, d_ref, h_ref, aux_ref, b_ref, x_ref):
    i = pl.program_id(0)
    acc = f_ref[0] + f_ref[1]
    den = d_ref[0] + d_ref[1]
    lane = lax.broadcasted_iota(jnp.int32, (RB, H), 1)
    aux = aux_ref[...]
    wself = jnp.sum(jnp.where(lane == 2, aux, 0.0), axis=1, keepdims=True)
    num = acc + wself * h_ref[...]
    xv = num / (den + wself + 1e-16) + b_ref[...]
    xv = jnp.maximum(xv, 0.0)
    rid = i * RB + lax.broadcasted_iota(jnp.int32, (RB, H), 0)
    x_ref[...] = jnp.where(rid < N, xv, 0.0)


def _tc_combine(feats, den, h, aux, b):
    return pl.pallas_call(
        _tc_combine_body,
        grid=(GRID,),
        in_specs=[
            pl.BlockSpec((2, RB, H), lambda i: (0, i, 0)),
            pl.BlockSpec((2, RB, 1), lambda i: (0, i, 0)),
            pl.BlockSpec((RB, H), lambda i: (i, 0)),
            pl.BlockSpec((RB, H), lambda i: (i, 0)),
            pl.BlockSpec((1, H), lambda i: (0, 0)),
        ],
        out_specs=pl.BlockSpec((RB, H), lambda i: (i, 0)),
        out_shape=jax.ShapeDtypeStruct((NPAD, H), jnp.float32),
    )(feats, den, h, aux, b.reshape(1, H))


FRB = 400


def _tc_final_body(x1_ref, x2_ref, x3_ref, w_ref, b_ref, o_ref):
    xjk = jnp.maximum(jnp.maximum(x1_ref[...], x2_ref[...]), x3_ref[...])
    o_ref[...] = (jnp.dot(xjk, w_ref[...], preferred_element_type=jnp.float32)
                  + b_ref[...])


def _tc_final(x1, x2, x3, w_out, b_out):
    return pl.pallas_call(
        _tc_final_body,
        grid=(N // FRB,),
        in_specs=[
            pl.BlockSpec((FRB, H), lambda i: (i, 0)),
            pl.BlockSpec((FRB, H), lambda i: (i, 0)),
            pl.BlockSpec((FRB, H), lambda i: (i, 0)),
            pl.BlockSpec((H, OUT), lambda i: (0, 0)),
            pl.BlockSpec((1, OUT), lambda i: (0, 0)),
        ],
        out_specs=pl.BlockSpec((FRB, OUT), lambda i: (i, 0)),
        out_shape=jax.ShapeDtypeStruct((N, OUT), jnp.float32),
    )(x1, x2, x3, w_out, b_out.reshape(1, OUT))


_SC_MESH = plsc.VectorSubcoreMesh(core_axis_name="c", subcore_axis_name="s")

_SPLAT_DNUMS = lax.GatherDimensionNumbers(
    offset_dims=(), collapsed_slice_dims=(0,), start_index_map=(0,))


def _splat(vec16, k):
    idx = jnp.full((16, 1), k, jnp.int32)
    return lax.gather(vec16, idx, _SPLAT_DNUMS, (1,),
                      mode=lax.GatherScatterMode.PROMISE_IN_BOUNDS)


@functools.partial(
    pl.kernel,
    mesh=_SC_MESH,
    out_type=jax.ShapeDtypeStruct((2, NROWS, H), jnp.float32),
    scratch_types=(
        [pltpu.VMEM((CH,), jnp.int32) for _ in range(8)]
        + [pltpu.VMEM((CH,), jnp.int32) for _ in range(8)]
        + [
            pltpu.VMEM((CH,), jnp.float32),
            pltpu.VMEM((CH,), jnp.float32),
            pltpu.VMEM((CH,), jnp.float32),
            pltpu.VMEM((CH,), jnp.float32),
            pltpu.VMEM((CH,), jnp.float32),
            pltpu.VMEM((CH,), jnp.float32),
            pltpu.VMEM((CH,), jnp.float32),
            pltpu.VMEM((CH,), jnp.float32),
            pltpu.VMEM((CH, H), jnp.float32),
            pltpu.VMEM((CH, H), jnp.float32),
            pltpu.VMEM((CH, H), jnp.float32),
            pltpu.VMEM((CH, H), jnp.float32),
            pltpu.VMEM((DEN_ROWS, H), jnp.float32),
            pltpu.VMEM((DEN_ROWS,), jnp.int32),
            pltpu.VMEM_SHARED((NROWS, H), jnp.float32),
            pltpu.SemaphoreType.DMA,
            pltpu.SemaphoreType.DMA,
            pltpu.SemaphoreType.DMA,
        ]
    ),
)
def _sc_edges(h_hbm, asrc_hbm, adst_hbm, src_hbm, dst_hbm, zeros_hbm,
              denidx_hbm, out_hbm,
              srcr0, srcr1, srcr2, srcr3, srcr4, srcr5, srcr6, srcr7,
              dstr0, dstr1, dstr2, dstr3, dstr4, dstr5, dstr6, dstr7,
              as0, as1, as2, as3, ad0, ad1, ad2, ad3,
              rows0, rows1, rows2, rows3,
              den_l, denidx_v, acc_sh, isem, gsem, ssem):
    c = lax.axis_index("c")
    s = lax.axis_index("s")
    wid = c * 16 + s
    src_v = (srcr0, srcr1, srcr2, srcr3, srcr4, srcr5, srcr6, srcr7)
    dst_v = (dstr0, dstr1, dstr2, dstr3, dstr4, dstr5, dstr6, dstr7)
    asv_v = (as0, as1, as2, as3)
    adv_v = (ad0, ad1, ad2, ad3)
    rows_v = (rows0, rows1, rows2, rows3)

    pltpu.sync_copy(zeros_hbm.at[pl.ds(s * RPT, RPT)],
                    acc_sh.at[pl.ds(s * RPT, RPT)])
    pltpu.sync_copy(zeros_hbm.at[pl.ds(0, DEN_ROWS)], den_l)
    pltpu.sync_copy(denidx_hbm, denidx_v)
    plsc.subcore_barrier()

    lane = lax.iota(jnp.int32, 16)

    def issue_idx(r, g):
        base = wid * PER_TILE + g * CH
        pltpu.async_copy(src_hbm.at[pl.ds(base, CH)], src_v[r], isem)
        pltpu.async_copy(dst_hbm.at[pl.ds(base, CH)], dst_v[r], isem)

    def wait_idx(r, g):
        base = wid * PER_TILE + g * CH
        pltpu.make_async_copy(src_hbm.at[pl.ds(base, CH)], src_v[r], isem).wait()
        pltpu.make_async_copy(dst_hbm.at[pl.ds(base, CH)], dst_v[r], isem).wait()

    def issue_gathers(r, b):
        pltpu.async_copy(h_hbm.at[src_v[r]], rows_v[b], gsem)
        pltpu.async_copy(asrc_hbm.at[src_v[r]], asv_v[b], gsem)
        pltpu.async_copy(adst_hbm.at[dst_v[r]], adv_v[b], gsem)

    def wait_gathers(r, b):
        pltpu.make_async_copy(h_hbm.at[src_v[r]], rows_v[b], gsem).wait()
        pltpu.make_async_copy(asrc_hbm.at[src_v[r]], asv_v[b], gsem).wait()
        pltpu.make_async_copy(adst_hbm.at[dst_v[r]], adv_v[b], gsem).wait()

    def wait_scatter(r, b):
        pltpu.make_async_copy(rows_v[b], acc_sh.at[dst_v[r]], ssem).wait()

    def compute_chunk(r, b):
        def group_body(t, carry):
            e = asv_v[b][pl.ds(t * 16, 16)] + adv_v[b][pl.ds(t * 16, 16)]
            e = jnp.where(e >= 0.0, e, 0.2 * e)
            w16 = jnp.exp(e)
            d16 = dst_v[r][pl.ds(t * 16, 16)]
            for k in range(16):
                rr_ = t * 16 + k
                wvec = jnp.full((16,), w16[k], jnp.float32)
                for j in range(H // 16):
                    rows_v[b][rr_, pl.ds(j * 16, 16)] = (
                        rows_v[b][rr_, pl.ds(j * 16, 16)] * wvec)
                d_s = d16[k]
                rr = lax.shift_right_logical(d_s, 7)
                bb = lax.bitwise_and(lax.shift_right_logical(d_s, 4), 7) * 16
                lk = lax.bitwise_and(d_s, 15)
                blk = den_l[rr, pl.ds(bb, 16)]
                den_l[rr, pl.ds(bb, 16)] = blk + jnp.where(lane == lk, wvec, 0.0)
            return carry

        lax.fori_loop(0, CH // 16, group_body, 0)

    def scatter_chunk(r, b):
        pltpu.async_copy(rows_v[b], acc_sh.at[dst_v[r]], ssem, add=True)

    for g0 in range(4):
        issue_idx(g0, g0)
    wait_idx(0, 0)
    issue_gathers(0, 0)
    wait_idx(1, 1)
    issue_gathers(1, 1)

    def octo_body(i, carry):
        for q in range(8):
            g = 8 * i + q
            r, b = q, q % 4

            @pl.when(g >= 2)
            def _():
                wait_scatter((q - 2) % 8, (q - 2) % 4)

            @pl.when(g + 4 < CHUNKS)
            def _():
                issue_idx((q + 4) % 8, g + 4)

            @pl.when(g + 2 < CHUNKS)
            def _():
                wait_idx((q + 2) % 8, g + 2)
                issue_gathers((q + 2) % 8, (q + 2) % 4)

            wait_gathers(r, b)
            compute_chunk(r, b)
            scatter_chunk(r, b)
        return carry

    lax.fori_loop(0, CHUNKS // 8, octo_body, 0)
    wait_scatter((CHUNKS - 2) % 8, (CHUNKS - 2) % 4)
    wait_scatter((CHUNKS - 1) % 8, (CHUNKS - 1) % 4)

    pltpu.async_copy(den_l, acc_sh.at[denidx_v], ssem, add=True)
    pltpu.make_async_copy(den_l, acc_sh.at[denidx_v], ssem).wait()
    plsc.subcore_barrier()

    pltpu.sync_copy(acc_sh.at[pl.ds(s * RPT, RPT)],
                    out_hbm.at[c, pl.ds(s * RPT, RPT)])


def kernel(x, edge_index, W1, a1_s, a1_d, b1, W2, a2_s, a2_d, b2,
           W3, a3_s, a3_d, b3, W_out, b_out):
    ei = edge_index.astype(jnp.int32)
    src = jnp.concatenate([ei[0], jnp.zeros((EPAD - E,), jnp.int32)])
    dst = jnp.concatenate([ei[1], jnp.full((EPAD - E,), DUMMY, jnp.int32)])
    xp = jnp.pad(x, ((0, NPAD - N), (0, 0)))
    zeros = jnp.zeros((NROWS, H), jnp.float32)
    denidx = NPAD + jnp.arange(DEN_ROWS, dtype=jnp.int32)

    def gat_layer(xin, W, a_s, a_d, b):
        h, aux = _tc_pre(xin, W, a_s, a_d)
        parts = _sc_edges(h, aux[:, 0], aux[:, 1], src, dst, zeros, denidx)
        feats = parts[:, :NPAD, :]
        den = parts[:, NPAD:NPAD + DEN_ROWS, :].reshape(2, NPAD, 1)
        return _tc_combine(feats, den, h, aux, b)

    x1 = gat_layer(xp, W1, a1_s, a1_d, b1)
    x2 = gat_layer(x1, W2, a2_s, a2_d, b2)
    x3 = gat_layer(x2, W3, a3_s, a3_d, b3)
    return _tc_final(x1, x2, x3, W_out, b_out)
